# baseline (speedup 1.0000x reference)
import functools

import jax
import jax.numpy as jnp
from jax import lax
from jax.experimental import pallas as pl
from jax.experimental.pallas import tpu as pltpu
from jax.experimental.pallas import tpu_sc as plsc

_INFO = plsc.get_sparse_core_info()
_NC = _INFO.num_cores
_NS = _INFO.num_subcores
_NW = _NC * _NS


def _embed_lookup(input_ids, table):
    b, h = input_ids.shape
    d = table.shape[1]
    rows_per_w = b // _NW
    mesh = plsc.VectorSubcoreMesh(core_axis_name="c", subcore_axis_name="s")

    @functools.partial(
        pl.kernel,
        mesh=mesh,
        compiler_params=pltpu.CompilerParams(use_tc_tiling_on_sc=False),
        out_type=jax.ShapeDtypeStruct((b, h, d), jnp.float32),
        scratch_types=[
            pltpu.VMEM((rows_per_w, h), jnp.int32),
            pltpu.VMEM((h, d), jnp.float32),
            pltpu.VMEM((h, d), jnp.float32),
            pltpu.SemaphoreType.DMA,
            pltpu.SemaphoreType.DMA,
            pltpu.SemaphoreType.DMA,
            pltpu.SemaphoreType.DMA,
        ],
    )
    def k(ids_hbm, table_hbm, out_hbm, idx_v, buf0, buf1, g0, g1, s0, s1):
        wid = lax.axis_index("s") * _NC + lax.axis_index("c")
        row0 = wid * rows_per_w
        pltpu.sync_copy(ids_hbm.at[pl.ds(row0, rows_per_w)], idx_v)

        def gather(i, buf, sem):
            pltpu.async_copy(table_hbm.at[idx_v.at[i]], buf, sem)

        def gather_wait(i, buf, sem):
            pltpu.make_async_copy(table_hbm.at[idx_v.at[i]], buf, sem).wait()

        def store(i, buf, sem):
            pltpu.async_copy(buf, out_hbm.at[row0 + i], sem)

        def store_wait(i, buf, sem):
            pltpu.make_async_copy(buf, out_hbm.at[row0 + i], sem).wait()

        gather(0, buf0, g0)
        gather(1, buf1, g1)

        def pair(p, _):
            i0 = 2 * p
            gather_wait(i0, buf0, g0)
            store(i0, buf0, s0)
            gather_wait(i0 + 1, buf1, g1)
            store(i0 + 1, buf1, s1)
            store_wait(i0, buf0, s0)
            gather(i0 + 2, buf0, g0)
            store_wait(i0 + 1, buf1, s1)
            gather(i0 + 3, buf1, g1)
            return 0

        lax.fori_loop(0, rows_per_w // 2 - 1, pair, 0)

        i0 = rows_per_w - 2
        gather_wait(i0, buf0, g0)
        store(i0, buf0, s0)
        gather_wait(i0 + 1, buf1, g1)
        store(i0 + 1, buf1, s1)
        store_wait(i0, buf0, s0)
        store_wait(i0 + 1, buf1, s1)

    return k(input_ids, table)


def kernel(input_ids, table):
    return _embed_lookup(input_ids.astype(jnp.int32), table)


# speedup vs baseline: 1.0250x; 1.0250x over previous
import functools

import jax
import jax.numpy as jnp
from jax import lax
from jax.experimental import pallas as pl
from jax.experimental.pallas import tpu as pltpu
from jax.experimental.pallas import tpu_sc as plsc

_INFO = plsc.get_sparse_core_info()
_NC = _INFO.num_cores
_NS = _INFO.num_subcores
_NW = _NC * _NS

_CHUNK = 512


def _embed_lookup(ids_flat, table):
    n = ids_flat.shape[0]
    d = table.shape[1]
    per_w = n // _NW
    n_chunks = per_w // _CHUNK
    mesh = plsc.VectorSubcoreMesh(core_axis_name="c", subcore_axis_name="s")

    @functools.partial(
        pl.kernel,
        mesh=mesh,
        compiler_params=pltpu.CompilerParams(use_tc_tiling_on_sc=False),
        out_type=jax.ShapeDtypeStruct((n, d), jnp.float32),
        scratch_types=[
            pltpu.VMEM((per_w,), jnp.int32),
            pltpu.VMEM((_CHUNK, d), jnp.float32),
            pltpu.VMEM((_CHUNK, d), jnp.float32),
            pltpu.SemaphoreType.DMA,
            pltpu.SemaphoreType.DMA,
            pltpu.SemaphoreType.DMA,
            pltpu.SemaphoreType.DMA,
        ],
    )
    def k(ids_hbm, table_hbm, out_hbm, idx_v, buf0, buf1, g0, g1, s0, s1):
        wid = lax.axis_index("s") * _NC + lax.axis_index("c")
        base = wid * per_w
        pltpu.sync_copy(ids_hbm.at[pl.ds(base, per_w)], idx_v)

        def gather(c, buf, sem):
            pltpu.async_copy(table_hbm.at[idx_v.at[pl.ds(c * _CHUNK, _CHUNK)]],
                             buf, sem)

        def gather_wait(c, buf, sem):
            pltpu.make_async_copy(
                table_hbm.at[idx_v.at[pl.ds(c * _CHUNK, _CHUNK)]],
                buf, sem).wait()

        def store(c, buf, sem):
            pltpu.async_copy(buf, out_hbm.at[pl.ds(base + c * _CHUNK, _CHUNK)],
                             sem)

        def store_wait(c, buf, sem):
            pltpu.make_async_copy(
                buf, out_hbm.at[pl.ds(base + c * _CHUNK, _CHUNK)],
                sem).wait()

        gather(0, buf0, g0)
        gather(1, buf1, g1)

        def pair(p, _):
            c0 = 2 * p
            gather_wait(c0, buf0, g0)
            store(c0, buf0, s0)
            gather_wait(c0 + 1, buf1, g1)
            store(c0 + 1, buf1, s1)
            store_wait(c0, buf0, s0)
            gather(c0 + 2, buf0, g0)
            store_wait(c0 + 1, buf1, s1)
            gather(c0 + 3, buf1, g1)
            return 0

        lax.fori_loop(0, n_chunks // 2 - 1, pair, 0)

        c0 = n_chunks - 2
        gather_wait(c0, buf0, g0)
        store(c0, buf0, s0)
        gather_wait(c0 + 1, buf1, g1)
        store(c0 + 1, buf1, s1)
        store_wait(c0, buf0, s0)
        store_wait(c0 + 1, buf1, s1)

    return k(ids_flat, table)


def kernel(input_ids, table):
    b, h = input_ids.shape
    out_flat = _embed_lookup(input_ids.astype(jnp.int32).reshape(-1), table)
    return out_flat.reshape(b, h, table.shape[1])


# speedup vs baseline: 1.0291x; 1.0040x over previous
import functools

import jax
import jax.numpy as jnp
from jax import lax
from jax.experimental import pallas as pl
from jax.experimental.pallas import tpu as pltpu
from jax.experimental.pallas import tpu_sc as plsc

_INFO = plsc.get_sparse_core_info()
_NC = _INFO.num_cores
_NS = _INFO.num_subcores
_NW = _NC * _NS

_CHUNK = 256
_NBUF = 4


def _embed_lookup(ids_flat, table):
    n = ids_flat.shape[0]
    d = table.shape[1]
    per_w = n // _NW
    n_chunks = per_w // _CHUNK
    n_rounds = n_chunks // _NBUF
    mesh = plsc.VectorSubcoreMesh(core_axis_name="c", subcore_axis_name="s")

    @functools.partial(
        pl.kernel,
        mesh=mesh,
        compiler_params=pltpu.CompilerParams(use_tc_tiling_on_sc=False),
        out_type=jax.ShapeDtypeStruct((n, d), jnp.float32),
        scratch_types=(
            [pltpu.VMEM((per_w,), jnp.int32)]
            + [pltpu.VMEM((_CHUNK, d), jnp.float32)] * _NBUF
            + [pltpu.SemaphoreType.DMA] * (2 * _NBUF)
        ),
    )
    def k(ids_hbm, table_hbm, out_hbm, idx_v, *bufs_sems):
        bufs = bufs_sems[:_NBUF]
        gsem = bufs_sems[_NBUF:2 * _NBUF]
        ssem = bufs_sems[2 * _NBUF:]
        wid = lax.axis_index("s") * _NC + lax.axis_index("c")
        base = wid * per_w
        pltpu.sync_copy(ids_hbm.at[pl.ds(base, per_w)], idx_v)

        def gather(c, j):
            pltpu.async_copy(table_hbm.at[idx_v.at[pl.ds(c * _CHUNK, _CHUNK)]],
                             bufs[j], gsem[j])

        def gather_wait(c, j):
            pltpu.make_async_copy(
                table_hbm.at[idx_v.at[pl.ds(c * _CHUNK, _CHUNK)]],
                bufs[j], gsem[j]).wait()

        def store(c, j):
            pltpu.async_copy(bufs[j],
                             out_hbm.at[pl.ds(base + c * _CHUNK, _CHUNK)],
                             ssem[j])

        def store_wait(c, j):
            pltpu.make_async_copy(
                bufs[j], out_hbm.at[pl.ds(base + c * _CHUNK, _CHUNK)],
                ssem[j]).wait()

        for j in range(_NBUF):
            gather(j, j)

        def round_(q, _):
            c = q * _NBUF
            for j in range(_NBUF):
                gather_wait(c + j, j)
                store(c + j, j)
            for j in range(_NBUF):
                store_wait(c + j, j)
                gather(c + _NBUF + j, j)
            return 0

        lax.fori_loop(0, n_rounds - 1, round_, 0)

        c = (n_rounds - 1) * _NBUF
        for j in range(_NBUF):
            gather_wait(c + j, j)
            store(c + j, j)
        for j in range(_NBUF):
            store_wait(c + j, j)

    return k(ids_flat, table)


def kernel(input_ids, table):
    b, h = input_ids.shape
    out_flat = _embed_lookup(input_ids.astype(jnp.int32).reshape(-1), table)
    return out_flat.reshape(b, h, table.shape[1])
